# baseline (speedup 1.0000x reference)
import functools

import jax
import jax.numpy as jnp
from jax import lax
from jax.experimental import pallas as pl
from jax.experimental.pallas import tpu as pltpu
from jax.experimental.pallas import tpu_sc as plsc

N = 10000
E = 320000
D = 128
C = 40

NW = 32
EPT = E // NW
CB = 80
NCH = EPT // CB
PCB = 125
PNCH = EPT // PCB
RPT = N // 16

_MESH = plsc.VectorSubcoreMesh(core_axis_name="c", subcore_axis_name="s")


@functools.partial(
    pl.kernel,
    out_type=jax.ShapeDtypeStruct((2, 16, RPT, 16), jnp.float32),
    mesh=_MESH,
    scratch_types=[
        pltpu.VMEM((NCH, CB), jnp.int32),
        pltpu.VMEM((CB, 16), jnp.float32),
        pltpu.VMEM((NCH, 16), jnp.float32),
        pltpu.VMEM_SHARED((N, 16), jnp.float32),
        pltpu.SemaphoreType.DMA,
    ],
)
def _deg_kernel(dstr_hbm, degp_hbm, dstidx, ones_v, z16, acc16, semd):
    c = lax.axis_index("c")
    s = lax.axis_index("s")
    wid = c * 16 + s

    one = jnp.full((16,), 1.0, jnp.float32)
    zero = jnp.zeros((16,), jnp.float32)

    def fill(i, _):
        ones_v[i, :] = one
        return 0

    lax.fori_loop(0, CB, fill, 0)

    def zfill(i, _):
        z16[i, :] = zero
        return 0

    lax.fori_loop(0, NCH, zfill, 0)

    for b in range(RPT // NCH):
        pltpu.sync_copy(z16, acc16.at[pl.ds(s * RPT + b * NCH, NCH)])
    plsc.subcore_barrier()

    pltpu.sync_copy(dstr_hbm.at[wid], dstidx)

    def body(j, _):
        pltpu.sync_copy(ones_v, acc16.at[dstidx.at[j]], add=True)
        return 0

    lax.fori_loop(0, NCH, body, 0)
    plsc.subcore_barrier()

    pltpu.sync_copy(acc16.at[pl.ds(s * RPT, RPT)], degp_hbm.at[c, s])


@functools.partial(
    pl.kernel,
    out_type=jax.ShapeDtypeStruct((2, 16, RPT, D), jnp.float32),
    mesh=_MESH,
    scratch_types=[
        pltpu.VMEM((4, 2, PCB), jnp.int32),
        pltpu.VMEM((2, PCB, D), jnp.float32),
        pltpu.VMEM_SHARED((N, D), jnp.float32),
        pltpu.SemaphoreType.DMA((2,)),
        pltpu.SemaphoreType.DMA((2,)),
        pltpu.SemaphoreType.DMA((2,)),
    ],
)
def _prop_kernel(g_hbm, ei_hbm, dummy_hbm, part_hbm,
                 idxb, rowsb, acc, semg, sems, semi):
    c = lax.axis_index("c")
    s = lax.axis_index("s")
    wid = c * 16 + s

    zero = jnp.zeros((16,), jnp.float32)

    def zfill(i, _):
        for j in range(D // 16):
            rowsb[0, i, pl.ds(j * 16, 16)] = zero
        return 0

    lax.fori_loop(0, PCB, zfill, 0)

    for b in range(RPT // PCB):
        pltpu.async_copy(rowsb.at[0], acc.at[pl.ds(s * RPT + b * PCB, PCB)],
                         semg.at[0])
    for b in range(RPT // PCB):
        pltpu.make_async_copy(rowsb.at[0], acc.at[pl.ds(0, PCB)],
                              semg.at[0]).wait()
    plsc.subcore_barrier()

    def wait_g(p):
        pltpu.make_async_copy(dummy_hbm, rowsb.at[p], semg.at[p]).wait()

    def wait_s(p):
        pltpu.make_async_copy(rowsb.at[p], acc.at[pl.ds(0, PCB)],
                              sems.at[p]).wait()

    def wait_i(p):
        pltpu.make_async_copy(ei_hbm.at[wid, 0], idxb.at[0], semi.at[p]).wait()

    def step(j, drain_prev, do_gather, do_idx):
        pj = lax.rem(j, 2)
        nx = 1 - pj
        if drain_prev:
            wait_s(nx)
        if do_gather:
            wait_i(nx)
            pltpu.async_copy(g_hbm.at[idxb.at[lax.rem(j + 1, 4), 0]],
                             rowsb.at[nx], semg.at[nx])
        if do_idx:
            pltpu.async_copy(ei_hbm.at[wid, j + 3],
                             idxb.at[lax.rem(j + 3, 4)], semi.at[nx])
        wait_g(pj)
        pltpu.async_copy(rowsb.at[pj], acc.at[idxb.at[lax.rem(j, 4), 1]],
                         sems.at[pj], add=True)

    pltpu.sync_copy(ei_hbm.at[wid, 0], idxb.at[0])
    pltpu.sync_copy(ei_hbm.at[wid, 1], idxb.at[1])
    pltpu.async_copy(g_hbm.at[idxb.at[0, 0]], rowsb.at[0], semg.at[0])
    pltpu.async_copy(ei_hbm.at[wid, 2], idxb.at[2], semi.at[0])
    pltpu.async_copy(ei_hbm.at[wid, 3], idxb.at[3], semi.at[1])
    pltpu.async_copy(g_hbm.at[idxb.at[1, 0]], rowsb.at[1], semg.at[1])
    wait_g(0)
    pltpu.async_copy(rowsb.at[0], acc.at[idxb.at[0, 1]], sems.at[0], add=True)

    def body(j, _):
        step(j, True, True, True)
        return 0

    lax.fori_loop(1, PNCH - 3, body, 0)
    step(PNCH - 3, True, True, False)
    step(PNCH - 2, True, True, False)
    step(PNCH - 1, True, False, False)
    wait_s((PNCH - 1) % 2)

    plsc.subcore_barrier()
    pltpu.sync_copy(acc.at[pl.ds(s * RPT, RPT)], part_hbm.at[c, s])


_BR = 1000


def _rinfo_body(degp_ref, rc_ref, r2c_ref):
    deg = degp_ref[0] + degp_ref[1]
    r = lax.rsqrt(deg)
    rc_ref[...] = r
    r2c_ref[...] = r * r


def _rinfo(degp):
    return pl.pallas_call(
        _rinfo_body,
        grid=(N // _BR,),
        in_specs=[pl.BlockSpec((2, _BR, 16), lambda i: (0, i, 0))],
        out_specs=[pl.BlockSpec((_BR, 16), lambda i: (i, 0)),
                   pl.BlockSpec((_BR, 16), lambda i: (i, 0))],
        out_shape=[jax.ShapeDtypeStruct((N, 16), jnp.float32),
                   jax.ShapeDtypeStruct((N, 16), jnp.float32)],
    )(degp)


def _bcast(ref16):
    return jnp.broadcast_to(ref16[:, 0:1], (ref16.shape[0], D))


def _lin_body(x_ref, wt_ref, b_ref, rc_ref, a0_ref, agg_ref, g_ref):
    h = jnp.dot(x_ref[...], wt_ref[...], preferred_element_type=jnp.float32)
    h = h + b_ref[...]
    agg_ref[...] = h * a0_ref[0, 0]
    g_ref[...] = h * _bcast(rc_ref[...])


def _lin(x, wt, b, rc, a0):
    return pl.pallas_call(
        _lin_body,
        grid=(N // _BR,),
        in_specs=[
            pl.BlockSpec((_BR, D), lambda i: (i, 0)),
            pl.BlockSpec((D, D), lambda i: (0, 0)),
            pl.BlockSpec((1, D), lambda i: (0, 0)),
            pl.BlockSpec((_BR, 16), lambda i: (i, 0)),
            pl.BlockSpec((1, 1), lambda i: (0, 0)),
        ],
        out_specs=[
            pl.BlockSpec((_BR, D), lambda i: (i, 0)),
            pl.BlockSpec((_BR, D), lambda i: (i, 0)),
        ],
        out_shape=[
            jax.ShapeDtypeStruct((N, D), jnp.float32),
            jax.ShapeDtypeStruct((N, D), jnp.float32),
        ],
    )(x, wt, b, rc, a0)


def _comb_body(agg_ref, p_ref, rc_ref, r2c_ref, ak_ref, aggo_ref, go_ref):
    t = p_ref[0] + p_ref[1]
    aggo_ref[...] = agg_ref[...] + ak_ref[0, 0] * (_bcast(rc_ref[...]) * t)
    go_ref[...] = _bcast(r2c_ref[...]) * t


def _comb(agg, p, rc, r2c, ak):
    return pl.pallas_call(
        _comb_body,
        grid=(N // _BR,),
        in_specs=[
            pl.BlockSpec((_BR, D), lambda i: (i, 0)),
            pl.BlockSpec((2, _BR, D), lambda i: (0, i, 0)),
            pl.BlockSpec((_BR, 16), lambda i: (i, 0)),
            pl.BlockSpec((_BR, 16), lambda i: (i, 0)),
            pl.BlockSpec((1, 1), lambda i: (0, 0)),
        ],
        out_specs=[pl.BlockSpec((_BR, D), lambda i: (i, 0)),
                   pl.BlockSpec((_BR, D), lambda i: (i, 0))],
        out_shape=[jax.ShapeDtypeStruct((N, D), jnp.float32),
                   jax.ShapeDtypeStruct((N, D), jnp.float32)],
    )(agg, p, rc, r2c, ak)


def _lin2_body(agg_ref, p_ref, rc_ref, ak_ref, wt_ref, b_ref, a0_ref,
               agg_ref_o, g_ref):
    t = p_ref[0] + p_ref[1]
    rb = _bcast(rc_ref[...])
    h = jnp.maximum(agg_ref[...] + ak_ref[0, 0] * (rb * t), 0.0)
    h = jnp.dot(h, wt_ref[...], preferred_element_type=jnp.float32) + b_ref[...]
    agg_ref_o[...] = h * a0_ref[0, 0]
    g_ref[...] = h * rb


def _lin2(agg, p, rc, ak, wt, b, a0):
    return pl.pallas_call(
        _lin2_body,
        grid=(N // _BR,),
        in_specs=[
            pl.BlockSpec((_BR, D), lambda i: (i, 0)),
            pl.BlockSpec((2, _BR, D), lambda i: (0, i, 0)),
            pl.BlockSpec((_BR, 16), lambda i: (i, 0)),
            pl.BlockSpec((1, 1), lambda i: (0, 0)),
            pl.BlockSpec((D, D), lambda i: (0, 0)),
            pl.BlockSpec((1, D), lambda i: (0, 0)),
            pl.BlockSpec((1, 1), lambda i: (0, 0)),
        ],
        out_specs=[
            pl.BlockSpec((_BR, D), lambda i: (i, 0)),
            pl.BlockSpec((_BR, D), lambda i: (i, 0)),
        ],
        out_shape=[
            jax.ShapeDtypeStruct((N, D), jnp.float32),
            jax.ShapeDtypeStruct((N, D), jnp.float32),
        ],
    )(agg, p, rc, ak, wt, b, a0)


def _final_body(agg_ref, p_ref, rc_ref, ak_ref, w2t_ref, b2_ref, o_ref):
    t = p_ref[0] + p_ref[1]
    h = jnp.maximum(agg_ref[...] + ak_ref[0, 0] * (_bcast(rc_ref[...]) * t), 0.0)
    logits = jnp.dot(h, w2t_ref[...], preferred_element_type=jnp.float32)
    logits = logits + b2_ref[...]
    mask = lax.broadcasted_iota(jnp.int32, logits.shape, 1) < C
    neg = jnp.where(mask, logits, -jnp.inf)
    m = jnp.max(neg, axis=1, keepdims=True)
    ex = jnp.where(mask, jnp.exp(logits - m), 0.0)
    ssum = jnp.sum(ex, axis=1, keepdims=True)
    o_ref[...] = logits - m - jnp.log(ssum)


def _final(agg, p, rc, ak, w2t, b2):
    return pl.pallas_call(
        _final_body,
        grid=(N // _BR,),
        in_specs=[
            pl.BlockSpec((_BR, D), lambda i: (i, 0)),
            pl.BlockSpec((2, _BR, D), lambda i: (0, i, 0)),
            pl.BlockSpec((_BR, 16), lambda i: (i, 0)),
            pl.BlockSpec((1, 1), lambda i: (0, 0)),
            pl.BlockSpec((D, D), lambda i: (0, 0)),
            pl.BlockSpec((1, D), lambda i: (0, 0)),
        ],
        out_specs=pl.BlockSpec((_BR, D), lambda i: (i, 0)),
        out_shape=jax.ShapeDtypeStruct((N, D), jnp.float32),
    )(agg, p, rc, ak, w2t, b2)


def kernel(x, edge_index, W0, b0, W1, b1, W2, b2, att):
    dst = edge_index[1].reshape(NW, NCH, CB)
    ei = jnp.stack([edge_index[0].reshape(NW, PNCH, PCB),
                    edge_index[1].reshape(NW, PNCH, PCB)], axis=2)
    dummy = jnp.zeros((PCB, D), jnp.float32)

    degp = _deg_kernel(dst).reshape(2, N, 16)
    rc, r2c = _rinfo(degp)

    w2t = jnp.zeros((D, D), jnp.float32).at[:, :C].set(W2.T)
    b2p = jnp.zeros((1, D), jnp.float32).at[0, :C].set(b2)

    agg, g = _lin(x, W0.T, b0.reshape(1, D), rc, att[0, 0].reshape(1, 1))
    for i in range(2):
        for k in range(1, 5):
            p = _prop_kernel(g, ei, dummy).reshape(2, N, D)
            ak = att[i, k].reshape(1, 1)
            if k < 4:
                agg, g = _comb(agg, p, rc, r2c, ak)
            elif i == 0:
                agg, g = _lin2(agg, p, rc, ak, W1.T, b1.reshape(1, D),
                               att[1, 0].reshape(1, 1))
            else:
                o = _final(agg, p, rc, ak, w2t, b2p)
    return o[:, :C]


# speedup vs baseline: 1.0042x; 1.0042x over previous
import functools

import jax
import jax.numpy as jnp
from jax import lax
from jax.experimental import pallas as pl
from jax.experimental.pallas import tpu as pltpu
from jax.experimental.pallas import tpu_sc as plsc

N = 10000
E = 320000
D = 128
C = 40

NW = 32
EPT = E // NW
CB = 80
NCH = EPT // CB
PCB = 125
PNCH = EPT // PCB
RPT = N // 16

_MESH = plsc.VectorSubcoreMesh(core_axis_name="c", subcore_axis_name="s")


@functools.partial(
    pl.kernel,
    out_type=jax.ShapeDtypeStruct((2, 16, RPT, 16), jnp.float32),
    mesh=_MESH,
    scratch_types=[
        pltpu.VMEM((NCH, CB), jnp.int32),
        pltpu.VMEM((CB, 16), jnp.float32),
        pltpu.VMEM((NCH, 16), jnp.float32),
        pltpu.VMEM_SHARED((N, 16), jnp.float32),
        pltpu.SemaphoreType.DMA,
    ],
)
def _deg_kernel(dstr_hbm, degp_hbm, dstidx, ones_v, z16, acc16, semd):
    c = lax.axis_index("c")
    s = lax.axis_index("s")
    wid = c * 16 + s

    one = jnp.full((16,), 1.0, jnp.float32)
    zero = jnp.zeros((16,), jnp.float32)

    def fill(i, _):
        ones_v[i, :] = one
        return 0

    lax.fori_loop(0, CB, fill, 0)

    def zfill(i, _):
        z16[i, :] = zero
        return 0

    lax.fori_loop(0, NCH, zfill, 0)

    for b in range(RPT // NCH):
        pltpu.sync_copy(z16, acc16.at[pl.ds(s * RPT + b * NCH, NCH)])
    plsc.subcore_barrier()

    pltpu.sync_copy(dstr_hbm.at[wid], dstidx)

    def body(j, _):
        pltpu.sync_copy(ones_v, acc16.at[dstidx.at[j]], add=True)
        return 0

    lax.fori_loop(0, NCH, body, 0)
    plsc.subcore_barrier()

    pltpu.sync_copy(acc16.at[pl.ds(s * RPT, RPT)], degp_hbm.at[c, s])


@functools.partial(
    pl.kernel,
    out_type=jax.ShapeDtypeStruct((2, 16, RPT, D), jnp.float32),
    mesh=_MESH,
    scratch_types=[
        pltpu.VMEM((4, 2, PCB), jnp.int32),
        pltpu.VMEM((2, PCB, D), jnp.float32),
        pltpu.VMEM_SHARED((N, D), jnp.float32),
        pltpu.SemaphoreType.DMA((2,)),
        pltpu.SemaphoreType.DMA((2,)),
        pltpu.SemaphoreType.DMA((2,)),
    ],
)
def _prop_kernel(g_hbm, ei_hbm, dummy_hbm, part_hbm,
                 idxb, rowsb, acc, semg, sems, semi):
    c = lax.axis_index("c")
    s = lax.axis_index("s")
    wid = c * 16 + s

    zero = jnp.zeros((16,), jnp.float32)

    def zfill(i, _):
        for j in range(D // 16):
            rowsb[0, i, pl.ds(j * 16, 16)] = zero
        return 0

    lax.fori_loop(0, PCB, zfill, 0)

    for b in range(RPT // PCB):
        pltpu.async_copy(rowsb.at[0], acc.at[pl.ds(s * RPT + b * PCB, PCB)],
                         semg.at[0])
    for b in range(RPT // PCB):
        pltpu.make_async_copy(rowsb.at[0], acc.at[pl.ds(0, PCB)],
                              semg.at[0]).wait()
    plsc.subcore_barrier()

    def wait_g(p):
        pltpu.make_async_copy(dummy_hbm, rowsb.at[p], semg.at[p]).wait()

    def wait_s(p):
        pltpu.make_async_copy(rowsb.at[p], acc.at[pl.ds(0, PCB)],
                              sems.at[p]).wait()

    def wait_i(p):
        pltpu.make_async_copy(ei_hbm.at[wid, 0], idxb.at[0], semi.at[p]).wait()

    def step(j, drain_prev, do_gather, do_idx):
        pj = lax.rem(j, 2)
        nx = 1 - pj
        if drain_prev:
            wait_s(nx)
        if do_gather:
            wait_i(nx)
            pltpu.async_copy(g_hbm.at[idxb.at[lax.rem(j + 1, 4), 0]],
                             rowsb.at[nx], semg.at[nx])
        if do_idx:
            pltpu.async_copy(ei_hbm.at[wid, j + 3],
                             idxb.at[lax.rem(j + 3, 4)], semi.at[nx])
        wait_g(pj)
        pltpu.async_copy(rowsb.at[pj], acc.at[idxb.at[lax.rem(j, 4), 1]],
                         sems.at[pj], add=True)

    pltpu.sync_copy(ei_hbm.at[wid, 0], idxb.at[0])
    pltpu.sync_copy(ei_hbm.at[wid, 1], idxb.at[1])
    pltpu.async_copy(g_hbm.at[idxb.at[0, 0]], rowsb.at[0], semg.at[0])
    pltpu.async_copy(ei_hbm.at[wid, 2], idxb.at[2], semi.at[0])
    pltpu.async_copy(ei_hbm.at[wid, 3], idxb.at[3], semi.at[1])
    pltpu.async_copy(g_hbm.at[idxb.at[1, 0]], rowsb.at[1], semg.at[1])
    wait_g(0)
    pltpu.async_copy(rowsb.at[0], acc.at[idxb.at[0, 1]], sems.at[0], add=True)

    def body(j, _):
        step(j, True, True, True)
        return 0

    lax.fori_loop(1, PNCH - 3, body, 0)
    step(PNCH - 3, True, True, False)
    step(PNCH - 2, True, True, False)
    step(PNCH - 1, True, False, False)
    wait_s((PNCH - 1) % 2)

    plsc.subcore_barrier()
    pltpu.sync_copy(acc.at[pl.ds(s * RPT, RPT)], part_hbm.at[c, s])


_BR = 1000


def _rinfo_body(degp_ref, r_ref):
    deg = degp_ref[0] + degp_ref[1]
    r = lax.rsqrt(deg[:, 0:1])
    r_ref[...] = jnp.broadcast_to(r, (_BR, D))


def _rinfo(degp):
    return pl.pallas_call(
        _rinfo_body,
        grid=(N // _BR,),
        in_specs=[pl.BlockSpec((2, _BR, 16), lambda i: (0, i, 0))],
        out_specs=pl.BlockSpec((_BR, D), lambda i: (i, 0)),
        out_shape=jax.ShapeDtypeStruct((N, D), jnp.float32),
    )(degp)


def _lin_body(x_ref, wt_ref, b_ref, r_ref, a0_ref, agg_ref, g_ref):
    h = jnp.dot(x_ref[...], wt_ref[...], preferred_element_type=jnp.float32)
    h = h + b_ref[...]
    agg_ref[...] = h * a0_ref[0, 0]
    g_ref[...] = h * r_ref[...]


def _lin(x, wt, b, rc, a0):
    return pl.pallas_call(
        _lin_body,
        grid=(N // _BR,),
        in_specs=[
            pl.BlockSpec((_BR, D), lambda i: (i, 0)),
            pl.BlockSpec((D, D), lambda i: (0, 0)),
            pl.BlockSpec((1, D), lambda i: (0, 0)),
            pl.BlockSpec((_BR, D), lambda i: (i, 0)),
            pl.BlockSpec((1, 1), lambda i: (0, 0)),
        ],
        out_specs=[
            pl.BlockSpec((_BR, D), lambda i: (i, 0)),
            pl.BlockSpec((_BR, D), lambda i: (i, 0)),
        ],
        out_shape=[
            jax.ShapeDtypeStruct((N, D), jnp.float32),
            jax.ShapeDtypeStruct((N, D), jnp.float32),
        ],
    )(x, wt, b, rc, a0)


def _comb_body(agg_ref, p_ref, r_ref, ak_ref, aggo_ref, go_ref):
    t = p_ref[0] + p_ref[1]
    r = r_ref[...]
    aggo_ref[...] = agg_ref[...] + ak_ref[0, 0] * (r * t)
    go_ref[...] = (r * r) * t


def _comb(agg, p, rc, ak):
    return pl.pallas_call(
        _comb_body,
        grid=(N // _BR,),
        in_specs=[
            pl.BlockSpec((_BR, D), lambda i: (i, 0)),
            pl.BlockSpec((2, _BR, D), lambda i: (0, i, 0)),
            pl.BlockSpec((_BR, D), lambda i: (i, 0)),
            pl.BlockSpec((1, 1), lambda i: (0, 0)),
        ],
        out_specs=[pl.BlockSpec((_BR, D), lambda i: (i, 0)),
                   pl.BlockSpec((_BR, D), lambda i: (i, 0))],
        out_shape=[jax.ShapeDtypeStruct((N, D), jnp.float32),
                   jax.ShapeDtypeStruct((N, D), jnp.float32)],
    )(agg, p, rc, ak)


def _lin2_body(agg_ref, p_ref, r_ref, ak_ref, wt_ref, b_ref, a0_ref,
               agg_ref_o, g_ref):
    t = p_ref[0] + p_ref[1]
    rb = r_ref[...]
    h = jnp.maximum(agg_ref[...] + ak_ref[0, 0] * (rb * t), 0.0)
    h = jnp.dot(h, wt_ref[...], preferred_element_type=jnp.float32) + b_ref[...]
    agg_ref_o[...] = h * a0_ref[0, 0]
    g_ref[...] = h * rb


def _lin2(agg, p, rc, ak, wt, b, a0):
    return pl.pallas_call(
        _lin2_body,
        grid=(N // _BR,),
        in_specs=[
            pl.BlockSpec((_BR, D), lambda i: (i, 0)),
            pl.BlockSpec((2, _BR, D), lambda i: (0, i, 0)),
            pl.BlockSpec((_BR, D), lambda i: (i, 0)),
            pl.BlockSpec((1, 1), lambda i: (0, 0)),
            pl.BlockSpec((D, D), lambda i: (0, 0)),
            pl.BlockSpec((1, D), lambda i: (0, 0)),
            pl.BlockSpec((1, 1), lambda i: (0, 0)),
        ],
        out_specs=[
            pl.BlockSpec((_BR, D), lambda i: (i, 0)),
            pl.BlockSpec((_BR, D), lambda i: (i, 0)),
        ],
        out_shape=[
            jax.ShapeDtypeStruct((N, D), jnp.float32),
            jax.ShapeDtypeStruct((N, D), jnp.float32),
        ],
    )(agg, p, rc, ak, wt, b, a0)


def _final_body(agg_ref, p_ref, r_ref, ak_ref, w2t_ref, b2_ref, o_ref):
    t = p_ref[0] + p_ref[1]
    h = jnp.maximum(agg_ref[...] + ak_ref[0, 0] * (r_ref[...] * t), 0.0)
    logits = jnp.dot(h, w2t_ref[...], preferred_element_type=jnp.float32)
    logits = logits + b2_ref[...]
    mask = lax.broadcasted_iota(jnp.int32, logits.shape, 1) < C
    neg = jnp.where(mask, logits, -jnp.inf)
    m = jnp.max(neg, axis=1, keepdims=True)
    ex = jnp.where(mask, jnp.exp(logits - m), 0.0)
    ssum = jnp.sum(ex, axis=1, keepdims=True)
    o_ref[...] = logits - m - jnp.log(ssum)


def _final(agg, p, rc, ak, w2t, b2):
    return pl.pallas_call(
        _final_body,
        grid=(N // _BR,),
        in_specs=[
            pl.BlockSpec((_BR, D), lambda i: (i, 0)),
            pl.BlockSpec((2, _BR, D), lambda i: (0, i, 0)),
            pl.BlockSpec((_BR, D), lambda i: (i, 0)),
            pl.BlockSpec((1, 1), lambda i: (0, 0)),
            pl.BlockSpec((D, D), lambda i: (0, 0)),
            pl.BlockSpec((1, D), lambda i: (0, 0)),
        ],
        out_specs=pl.BlockSpec((_BR, D), lambda i: (i, 0)),
        out_shape=jax.ShapeDtypeStruct((N, D), jnp.float32),
    )(agg, p, rc, ak, w2t, b2)


def kernel(x, edge_index, W0, b0, W1, b1, W2, b2, att):
    dst = edge_index[1].reshape(NW, NCH, CB)
    ei = jnp.stack([edge_index[0].reshape(NW, PNCH, PCB),
                    edge_index[1].reshape(NW, PNCH, PCB)], axis=2)
    dummy = jnp.zeros((PCB, D), jnp.float32)

    degp = _deg_kernel(dst).reshape(2, N, 16)
    rc = _rinfo(degp)

    w2t = jnp.zeros((D, D), jnp.float32).at[:, :C].set(W2.T)
    b2p = jnp.zeros((1, D), jnp.float32).at[0, :C].set(b2)

    agg, g = _lin(x, W0.T, b0.reshape(1, D), rc, att[0, 0].reshape(1, 1))
    for i in range(2):
        for k in range(1, 5):
            p = _prop_kernel(g, ei, dummy).reshape(2, N, D)
            ak = att[i, k].reshape(1, 1)
            if k < 4:
                agg, g = _comb(agg, p, rc, ak)
            elif i == 0:
                agg, g = _lin2(agg, p, rc, ak, W1.T, b1.reshape(1, D),
                               att[1, 0].reshape(1, 1))
            else:
                o = _final(agg, p, rc, ak, w2t, b2p)
    return o[:, :C]


# speedup vs baseline: 1.0079x; 1.0037x over previous
import functools

import jax
import jax.numpy as jnp
from jax import lax
from jax.experimental import pallas as pl
from jax.experimental.pallas import tpu as pltpu
from jax.experimental.pallas import tpu_sc as plsc

N = 10000
E = 320000
D = 128
C = 40

NW = 32
EPT = E // NW
CB = 80
NCH = EPT // CB
PCB = 125
PNCH = EPT // PCB
RPT = N // 16

_MESH = plsc.VectorSubcoreMesh(core_axis_name="c", subcore_axis_name="s")


@functools.partial(
    pl.kernel,
    out_type=jax.ShapeDtypeStruct((2, 16, RPT, 16), jnp.float32),
    mesh=_MESH,
    scratch_types=[
        pltpu.VMEM((NCH, CB), jnp.int32),
        pltpu.VMEM((CB, 16), jnp.float32),
        pltpu.VMEM((NCH, 16), jnp.float32),
        pltpu.VMEM_SHARED((N, 16), jnp.float32),
        pltpu.SemaphoreType.DMA,
    ],
)
def _deg_kernel(dstr_hbm, degp_hbm, dstidx, ones_v, z16, acc16, semd):
    c = lax.axis_index("c")
    s = lax.axis_index("s")
    wid = c * 16 + s

    one = jnp.full((16,), 1.0, jnp.float32)
    zero = jnp.zeros((16,), jnp.float32)

    def fill(i, _):
        ones_v[i, :] = one
        return 0

    lax.fori_loop(0, CB, fill, 0)

    def zfill(i, _):
        z16[i, :] = zero
        return 0

    lax.fori_loop(0, NCH, zfill, 0)

    for b in range(RPT // NCH):
        pltpu.sync_copy(z16, acc16.at[pl.ds(s * RPT + b * NCH, NCH)])
    plsc.subcore_barrier()

    pltpu.sync_copy(dstr_hbm.at[wid], dstidx)

    def body(j, _):
        pltpu.sync_copy(ones_v, acc16.at[dstidx.at[j]], add=True)
        return 0

    lax.fori_loop(0, NCH, body, 0)
    plsc.subcore_barrier()

    pltpu.sync_copy(acc16.at[pl.ds(s * RPT, RPT)], degp_hbm.at[c, s])


@functools.partial(
    pl.kernel,
    out_type=jax.ShapeDtypeStruct((2, 16, RPT, D), jnp.float32),
    mesh=_MESH,
    scratch_types=[
        pltpu.VMEM((4, 2, PCB), jnp.int32),
        pltpu.VMEM((2, PCB, D), jnp.float32),
        pltpu.VMEM_SHARED((N, D), jnp.float32),
        pltpu.SemaphoreType.DMA((2,)),
        pltpu.SemaphoreType.DMA((2,)),
        pltpu.SemaphoreType.DMA((2,)),
    ],
)
def _prop_kernel(g_hbm, ei_hbm, dummy_hbm, part_hbm,
                 idxb, rowsb, acc, semg, sems, semi):
    c = lax.axis_index("c")
    s = lax.axis_index("s")
    wid = c * 16 + s

    zero = jnp.zeros((16,), jnp.float32)

    def zfill(i, _):
        for j in range(D // 16):
            rowsb[0, i, pl.ds(j * 16, 16)] = zero
        return 0

    lax.fori_loop(0, PCB, zfill, 0)

    for b in range(RPT // PCB):
        pltpu.async_copy(rowsb.at[0], acc.at[pl.ds(s * RPT + b * PCB, PCB)],
                         semg.at[0])
    for b in range(RPT // PCB):
        pltpu.make_async_copy(rowsb.at[0], acc.at[pl.ds(0, PCB)],
                              semg.at[0]).wait()
    plsc.subcore_barrier()

    def wait_g(p):
        pltpu.make_async_copy(dummy_hbm, rowsb.at[p], semg.at[p]).wait()

    def wait_s(p):
        pltpu.make_async_copy(rowsb.at[p], acc.at[pl.ds(0, PCB)],
                              sems.at[p]).wait()

    def wait_i(p):
        pltpu.make_async_copy(ei_hbm.at[wid, 0], idxb.at[0], semi.at[p]).wait()

    def gather2(slot, nx):
        pltpu.async_copy(g_hbm.at[idxb.at[slot, 0, pl.ds(0, 64)]],
                         rowsb.at[nx, pl.ds(0, 64)], semg.at[nx])
        pltpu.async_copy(g_hbm.at[idxb.at[slot, 0, pl.ds(64, PCB - 64)]],
                         rowsb.at[nx, pl.ds(64, PCB - 64)], semg.at[nx])

    def step(j, drain_prev, do_gather, do_idx):
        pj = lax.rem(j, 2)
        nx = 1 - pj
        if drain_prev:
            wait_s(nx)
        if do_gather:
            wait_i(nx)
            gather2(lax.rem(j + 1, 4), nx)
        if do_idx:
            pltpu.async_copy(ei_hbm.at[wid, j + 3],
                             idxb.at[lax.rem(j + 3, 4)], semi.at[nx])
        wait_g(pj)
        pltpu.async_copy(rowsb.at[pj], acc.at[idxb.at[lax.rem(j, 4), 1]],
                         sems.at[pj], add=True)

    pltpu.sync_copy(ei_hbm.at[wid, 0], idxb.at[0])
    pltpu.sync_copy(ei_hbm.at[wid, 1], idxb.at[1])
    gather2(0, 0)
    pltpu.async_copy(ei_hbm.at[wid, 2], idxb.at[2], semi.at[0])
    pltpu.async_copy(ei_hbm.at[wid, 3], idxb.at[3], semi.at[1])
    gather2(1, 1)
    wait_g(0)
    pltpu.async_copy(rowsb.at[0], acc.at[idxb.at[0, 1]], sems.at[0], add=True)

    def body(j, _):
        step(j, True, True, True)
        return 0

    lax.fori_loop(1, PNCH - 3, body, 0)
    step(PNCH - 3, True, True, False)
    step(PNCH - 2, True, True, False)
    step(PNCH - 1, True, False, False)
    wait_s((PNCH - 1) % 2)

    plsc.subcore_barrier()
    pltpu.sync_copy(acc.at[pl.ds(s * RPT, RPT)], part_hbm.at[c, s])


_BR = 1000


def _rinfo_body(degp_ref, r_ref):
    deg = degp_ref[0] + degp_ref[1]
    r = lax.rsqrt(deg[:, 0:1])
    r_ref[...] = jnp.broadcast_to(r, (_BR, D))


def _rinfo(degp):
    return pl.pallas_call(
        _rinfo_body,
        grid=(N // _BR,),
        in_specs=[pl.BlockSpec((2, _BR, 16), lambda i: (0, i, 0))],
        out_specs=pl.BlockSpec((_BR, D), lambda i: (i, 0)),
        out_shape=jax.ShapeDtypeStruct((N, D), jnp.float32),
    )(degp)


def _lin_body(x_ref, wt_ref, b_ref, r_ref, a0_ref, agg_ref, g_ref):
    h = jnp.dot(x_ref[...], wt_ref[...], preferred_element_type=jnp.float32)
    h = h + b_ref[...]
    agg_ref[...] = h * a0_ref[0, 0]
    g_ref[...] = h * r_ref[...]


def _lin(x, wt, b, rc, a0):
    return pl.pallas_call(
        _lin_body,
        grid=(N // _BR,),
        in_specs=[
            pl.BlockSpec((_BR, D), lambda i: (i, 0)),
            pl.BlockSpec((D, D), lambda i: (0, 0)),
            pl.BlockSpec((1, D), lambda i: (0, 0)),
            pl.BlockSpec((_BR, D), lambda i: (i, 0)),
            pl.BlockSpec((1, 1), lambda i: (0, 0)),
        ],
        out_specs=[
            pl.BlockSpec((_BR, D), lambda i: (i, 0)),
            pl.BlockSpec((_BR, D), lambda i: (i, 0)),
        ],
        out_shape=[
            jax.ShapeDtypeStruct((N, D), jnp.float32),
            jax.ShapeDtypeStruct((N, D), jnp.float32),
        ],
    )(x, wt, b, rc, a0)


def _comb_body(agg_ref, p_ref, r_ref, ak_ref, aggo_ref, go_ref):
    t = p_ref[0] + p_ref[1]
    r = r_ref[...]
    aggo_ref[...] = agg_ref[...] + ak_ref[0, 0] * (r * t)
    go_ref[...] = (r * r) * t


def _comb(agg, p, rc, ak):
    return pl.pallas_call(
        _comb_body,
        grid=(N // _BR,),
        in_specs=[
            pl.BlockSpec((_BR, D), lambda i: (i, 0)),
            pl.BlockSpec((2, _BR, D), lambda i: (0, i, 0)),
            pl.BlockSpec((_BR, D), lambda i: (i, 0)),
            pl.BlockSpec((1, 1), lambda i: (0, 0)),
        ],
        out_specs=[pl.BlockSpec((_BR, D), lambda i: (i, 0)),
                   pl.BlockSpec((_BR, D), lambda i: (i, 0))],
        out_shape=[jax.ShapeDtypeStruct((N, D), jnp.float32),
                   jax.ShapeDtypeStruct((N, D), jnp.float32)],
    )(agg, p, rc, ak)


def _lin2_body(agg_ref, p_ref, r_ref, ak_ref, wt_ref, b_ref, a0_ref,
               agg_ref_o, g_ref):
    t = p_ref[0] + p_ref[1]
    rb = r_ref[...]
    h = jnp.maximum(agg_ref[...] + ak_ref[0, 0] * (rb * t), 0.0)
    h = jnp.dot(h, wt_ref[...], preferred_element_type=jnp.float32) + b_ref[...]
    agg_ref_o[...] = h * a0_ref[0, 0]
    g_ref[...] = h * rb


def _lin2(agg, p, rc, ak, wt, b, a0):
    return pl.pallas_call(
        _lin2_body,
        grid=(N // _BR,),
        in_specs=[
            pl.BlockSpec((_BR, D), lambda i: (i, 0)),
            pl.BlockSpec((2, _BR, D), lambda i: (0, i, 0)),
            pl.BlockSpec((_BR, D), lambda i: (i, 0)),
            pl.BlockSpec((1, 1), lambda i: (0, 0)),
            pl.BlockSpec((D, D), lambda i: (0, 0)),
            pl.BlockSpec((1, D), lambda i: (0, 0)),
            pl.BlockSpec((1, 1), lambda i: (0, 0)),
        ],
        out_specs=[
            pl.BlockSpec((_BR, D), lambda i: (i, 0)),
            pl.BlockSpec((_BR, D), lambda i: (i, 0)),
        ],
        out_shape=[
            jax.ShapeDtypeStruct((N, D), jnp.float32),
            jax.ShapeDtypeStruct((N, D), jnp.float32),
        ],
    )(agg, p, rc, ak, wt, b, a0)


def _final_body(agg_ref, p_ref, r_ref, ak_ref, w2t_ref, b2_ref, o_ref):
    t = p_ref[0] + p_ref[1]
    h = jnp.maximum(agg_ref[...] + ak_ref[0, 0] * (r_ref[...] * t), 0.0)
    logits = jnp.dot(h, w2t_ref[...], preferred_element_type=jnp.float32)
    logits = logits + b2_ref[...]
    mask = lax.broadcasted_iota(jnp.int32, logits.shape, 1) < C
    neg = jnp.where(mask, logits, -jnp.inf)
    m = jnp.max(neg, axis=1, keepdims=True)
    ex = jnp.where(mask, jnp.exp(logits - m), 0.0)
    ssum = jnp.sum(ex, axis=1, keepdims=True)
    o_ref[...] = logits - m - jnp.log(ssum)


def _final(agg, p, rc, ak, w2t, b2):
    return pl.pallas_call(
        _final_body,
        grid=(N // _BR,),
        in_specs=[
            pl.BlockSpec((_BR, D), lambda i: (i, 0)),
            pl.BlockSpec((2, _BR, D), lambda i: (0, i, 0)),
            pl.BlockSpec((_BR, D), lambda i: (i, 0)),
            pl.BlockSpec((1, 1), lambda i: (0, 0)),
            pl.BlockSpec((D, D), lambda i: (0, 0)),
            pl.BlockSpec((1, D), lambda i: (0, 0)),
        ],
        out_specs=pl.BlockSpec((_BR, D), lambda i: (i, 0)),
        out_shape=jax.ShapeDtypeStruct((N, D), jnp.float32),
    )(agg, p, rc, ak, w2t, b2)


def kernel(x, edge_index, W0, b0, W1, b1, W2, b2, att):
    dst = edge_index[1].reshape(NW, NCH, CB)
    ei = jnp.stack([edge_index[0].reshape(NW, PNCH, PCB),
                    edge_index[1].reshape(NW, PNCH, PCB)], axis=2)
    dummy = jnp.zeros((PCB, D), jnp.float32)

    degp = _deg_kernel(dst).reshape(2, N, 16)
    rc = _rinfo(degp)

    w2t = jnp.zeros((D, D), jnp.float32).at[:, :C].set(W2.T)
    b2p = jnp.zeros((1, D), jnp.float32).at[0, :C].set(b2)

    agg, g = _lin(x, W0.T, b0.reshape(1, D), rc, att[0, 0].reshape(1, 1))
    for i in range(2):
        for k in range(1, 5):
            p = _prop_kernel(g, ei, dummy).reshape(2, N, D)
            ak = att[i, k].reshape(1, 1)
            if k < 4:
                agg, g = _comb(agg, p, rc, ak)
            elif i == 0:
                agg, g = _lin2(agg, p, rc, ak, W1.T, b1.reshape(1, D),
                               att[1, 0].reshape(1, 1))
            else:
                o = _final(agg, p, rc, ak, w2t, b2p)
    return o[:, :C]


# speedup vs baseline: 1.0121x; 1.0042x over previous
import functools

import jax
import jax.numpy as jnp
from jax import lax
from jax.experimental import pallas as pl
from jax.experimental.pallas import tpu as pltpu
from jax.experimental.pallas import tpu_sc as plsc

N = 10000
E = 320000
D = 128
C = 40

NW = 32
EPT = E // NW
CB = 80
NCH = EPT // CB
PCB = 125
PNCH = EPT // PCB
RPT = N // 16

_MESH = plsc.VectorSubcoreMesh(core_axis_name="c", subcore_axis_name="s")


@functools.partial(
    pl.kernel,
    out_type=jax.ShapeDtypeStruct((2, 16, RPT, 16), jnp.float32),
    mesh=_MESH,
    scratch_types=[
        pltpu.VMEM((NCH, CB), jnp.int32),
        pltpu.VMEM((CB, 16), jnp.float32),
        pltpu.VMEM((NCH, 16), jnp.float32),
        pltpu.VMEM_SHARED((N, 16), jnp.float32),
        pltpu.SemaphoreType.DMA,
    ],
)
def _deg_kernel(dstr_hbm, degp_hbm, dstidx, ones_v, z16, acc16, semd):
    c = lax.axis_index("c")
    s = lax.axis_index("s")
    wid = c * 16 + s

    one = jnp.full((16,), 1.0, jnp.float32)
    zero = jnp.zeros((16,), jnp.float32)

    def fill(i, _):
        ones_v[i, :] = one
        return 0

    lax.fori_loop(0, CB, fill, 0)

    def zfill(i, _):
        z16[i, :] = zero
        return 0

    lax.fori_loop(0, NCH, zfill, 0)

    for b in range(RPT // NCH):
        pltpu.sync_copy(z16, acc16.at[pl.ds(s * RPT + b * NCH, NCH)])
    plsc.subcore_barrier()

    pltpu.sync_copy(dstr_hbm.at[wid], dstidx)

    def body(j, _):
        pltpu.sync_copy(ones_v, acc16.at[dstidx.at[j]], add=True)
        return 0

    lax.fori_loop(0, NCH, body, 0)
    plsc.subcore_barrier()

    pltpu.sync_copy(acc16.at[pl.ds(s * RPT, RPT)], degp_hbm.at[c, s])


@functools.partial(
    pl.kernel,
    out_type=jax.ShapeDtypeStruct((2, 16, RPT, D), jnp.float32),
    mesh=_MESH,
    scratch_types=[
        pltpu.VMEM((4, 2, PCB), jnp.int32),
        pltpu.VMEM((2, PCB, D), jnp.float32),
        pltpu.VMEM_SHARED((N, D), jnp.float32),
        pltpu.SemaphoreType.DMA((2,)),
        pltpu.SemaphoreType.DMA((2,)),
        pltpu.SemaphoreType.DMA((2,)),
    ],
)
def _prop_kernel(g_hbm, ei_hbm, dummy_hbm, part_hbm,
                 idxb, rowsb, acc, semg, sems, semi):
    c = lax.axis_index("c")
    s = lax.axis_index("s")
    wid = c * 16 + s

    zero = jnp.zeros((16,), jnp.float32)

    def zfill(i, _):
        for j in range(D // 16):
            rowsb[0, i, pl.ds(j * 16, 16)] = zero
        return 0

    lax.fori_loop(0, PCB, zfill, 0)

    for b in range(RPT // PCB):
        pltpu.async_copy(rowsb.at[0], acc.at[pl.ds(s * RPT + b * PCB, PCB)],
                         semg.at[0])
    for b in range(RPT // PCB):
        pltpu.make_async_copy(rowsb.at[0], acc.at[pl.ds(0, PCB)],
                              semg.at[0]).wait()
    plsc.subcore_barrier()

    def wait_g(p):
        pltpu.make_async_copy(dummy_hbm, rowsb.at[p], semg.at[p]).wait()

    def wait_s(p):
        pltpu.make_async_copy(rowsb.at[p], acc.at[pl.ds(0, PCB)],
                              sems.at[p]).wait()

    def wait_i(p):
        pltpu.make_async_copy(ei_hbm.at[wid, 0], idxb.at[0], semi.at[p]).wait()

    def gather2(slot, nx):
        pltpu.async_copy(g_hbm.at[idxb.at[slot, 0, pl.ds(0, 64)]],
                         rowsb.at[nx, pl.ds(0, 64)], semg.at[nx])
        pltpu.async_copy(g_hbm.at[idxb.at[slot, 0, pl.ds(64, PCB - 64)]],
                         rowsb.at[nx, pl.ds(64, PCB - 64)], semg.at[nx])

    def step(j, drain_prev, do_gather, do_idx):
        pj = lax.rem(j, 2)
        nx = 1 - pj
        if drain_prev:
            wait_s(nx)
        if do_gather:
            wait_i(nx)
            gather2(lax.rem(j + 1, 4), nx)
        if do_idx:
            pltpu.async_copy(ei_hbm.at[wid, j + 3],
                             idxb.at[lax.rem(j + 3, 4)], semi.at[nx])
        wait_g(pj)
        pltpu.async_copy(rowsb.at[pj], acc.at[idxb.at[lax.rem(j, 4), 1]],
                         sems.at[pj], add=True)

    pltpu.sync_copy(ei_hbm.at[wid, 0], idxb.at[0])
    pltpu.sync_copy(ei_hbm.at[wid, 1], idxb.at[1])
    gather2(0, 0)
    pltpu.async_copy(ei_hbm.at[wid, 2], idxb.at[2], semi.at[0])
    pltpu.async_copy(ei_hbm.at[wid, 3], idxb.at[3], semi.at[1])
    gather2(1, 1)
    wait_g(0)
    pltpu.async_copy(rowsb.at[0], acc.at[idxb.at[0, 1]], sems.at[0], add=True)

    def body(j, _):
        step(j, True, True, True)
        return 0

    lax.fori_loop(1, PNCH - 3, body, 0)
    step(PNCH - 3, True, True, False)
    step(PNCH - 2, True, True, False)
    step(PNCH - 1, True, False, False)
    wait_s((PNCH - 1) % 2)

    plsc.subcore_barrier()
    pltpu.sync_copy(acc.at[pl.ds(s * RPT, RPT)], part_hbm.at[c, s])


_BR = 1000


def _lin_body(x_ref, wt_ref, b_ref, degp_ref, a0_ref, agg_ref, g_ref, r_ref):
    deg = degp_ref[0] + degp_ref[1]
    r = jnp.broadcast_to(lax.rsqrt(deg[:, 0:1]), (_BR, D))
    r_ref[...] = r
    h = jnp.dot(x_ref[...], wt_ref[...], preferred_element_type=jnp.float32)
    h = h + b_ref[...]
    agg_ref[...] = h * a0_ref[0, 0]
    g_ref[...] = h * r


def _lin(x, wt, b, degp, a0):
    return pl.pallas_call(
        _lin_body,
        grid=(N // _BR,),
        in_specs=[
            pl.BlockSpec((_BR, D), lambda i: (i, 0)),
            pl.BlockSpec((D, D), lambda i: (0, 0)),
            pl.BlockSpec((1, D), lambda i: (0, 0)),
            pl.BlockSpec((2, _BR, 16), lambda i: (0, i, 0)),
            pl.BlockSpec((1, 1), lambda i: (0, 0)),
        ],
        out_specs=[
            pl.BlockSpec((_BR, D), lambda i: (i, 0)),
            pl.BlockSpec((_BR, D), lambda i: (i, 0)),
            pl.BlockSpec((_BR, D), lambda i: (i, 0)),
        ],
        out_shape=[
            jax.ShapeDtypeStruct((N, D), jnp.float32),
            jax.ShapeDtypeStruct((N, D), jnp.float32),
            jax.ShapeDtypeStruct((N, D), jnp.float32),
        ],
    )(x, wt, b, degp, a0)


def _comb_body(agg_ref, p_ref, r_ref, ak_ref, aggo_ref, go_ref):
    t = p_ref[0] + p_ref[1]
    r = r_ref[...]
    aggo_ref[...] = agg_ref[...] + ak_ref[0, 0] * (r * t)
    go_ref[...] = (r * r) * t


def _comb(agg, p, rc, ak):
    return pl.pallas_call(
        _comb_body,
        grid=(N // _BR,),
        in_specs=[
            pl.BlockSpec((_BR, D), lambda i: (i, 0)),
            pl.BlockSpec((2, _BR, D), lambda i: (0, i, 0)),
            pl.BlockSpec((_BR, D), lambda i: (i, 0)),
            pl.BlockSpec((1, 1), lambda i: (0, 0)),
        ],
        out_specs=[pl.BlockSpec((_BR, D), lambda i: (i, 0)),
                   pl.BlockSpec((_BR, D), lambda i: (i, 0))],
        out_shape=[jax.ShapeDtypeStruct((N, D), jnp.float32),
                   jax.ShapeDtypeStruct((N, D), jnp.float32)],
    )(agg, p, rc, ak)


def _lin2_body(agg_ref, p_ref, r_ref, ak_ref, wt_ref, b_ref, a0_ref,
               agg_ref_o, g_ref):
    t = p_ref[0] + p_ref[1]
    rb = r_ref[...]
    h = jnp.maximum(agg_ref[...] + ak_ref[0, 0] * (rb * t), 0.0)
    h = jnp.dot(h, wt_ref[...], preferred_element_type=jnp.float32) + b_ref[...]
    agg_ref_o[...] = h * a0_ref[0, 0]
    g_ref[...] = h * rb


def _lin2(agg, p, rc, ak, wt, b, a0):
    return pl.pallas_call(
        _lin2_body,
        grid=(N // _BR,),
        in_specs=[
            pl.BlockSpec((_BR, D), lambda i: (i, 0)),
            pl.BlockSpec((2, _BR, D), lambda i: (0, i, 0)),
            pl.BlockSpec((_BR, D), lambda i: (i, 0)),
            pl.BlockSpec((1, 1), lambda i: (0, 0)),
            pl.BlockSpec((D, D), lambda i: (0, 0)),
            pl.BlockSpec((1, D), lambda i: (0, 0)),
            pl.BlockSpec((1, 1), lambda i: (0, 0)),
        ],
        out_specs=[
            pl.BlockSpec((_BR, D), lambda i: (i, 0)),
            pl.BlockSpec((_BR, D), lambda i: (i, 0)),
        ],
        out_shape=[
            jax.ShapeDtypeStruct((N, D), jnp.float32),
            jax.ShapeDtypeStruct((N, D), jnp.float32),
        ],
    )(agg, p, rc, ak, wt, b, a0)


def _final_body(agg_ref, p_ref, r_ref, ak_ref, w2t_ref, b2_ref, o_ref):
    t = p_ref[0] + p_ref[1]
    h = jnp.maximum(agg_ref[...] + ak_ref[0, 0] * (r_ref[...] * t), 0.0)
    logits = jnp.dot(h, w2t_ref[...], preferred_element_type=jnp.float32)
    logits = logits + b2_ref[...]
    mask = lax.broadcasted_iota(jnp.int32, logits.shape, 1) < C
    neg = jnp.where(mask, logits, -jnp.inf)
    m = jnp.max(neg, axis=1, keepdims=True)
    ex = jnp.where(mask, jnp.exp(logits - m), 0.0)
    ssum = jnp.sum(ex, axis=1, keepdims=True)
    o_ref[...] = logits - m - jnp.log(ssum)


def _final(agg, p, rc, ak, w2t, b2):
    return pl.pallas_call(
        _final_body,
        grid=(N // _BR,),
        in_specs=[
            pl.BlockSpec((_BR, D), lambda i: (i, 0)),
            pl.BlockSpec((2, _BR, D), lambda i: (0, i, 0)),
            pl.BlockSpec((_BR, D), lambda i: (i, 0)),
            pl.BlockSpec((1, 1), lambda i: (0, 0)),
            pl.BlockSpec((D, D), lambda i: (0, 0)),
            pl.BlockSpec((1, D), lambda i: (0, 0)),
        ],
        out_specs=pl.BlockSpec((_BR, D), lambda i: (i, 0)),
        out_shape=jax.ShapeDtypeStruct((N, D), jnp.float32),
    )(agg, p, rc, ak, w2t, b2)


def kernel(x, edge_index, W0, b0, W1, b1, W2, b2, att):
    dst = edge_index[1].reshape(NW, NCH, CB)
    ei = jnp.stack([edge_index[0].reshape(NW, PNCH, PCB),
                    edge_index[1].reshape(NW, PNCH, PCB)], axis=2)
    dummy = jnp.zeros((PCB, D), jnp.float32)

    degp = _deg_kernel(dst).reshape(2, N, 16)

    w2t = jnp.zeros((D, D), jnp.float32).at[:, :C].set(W2.T)
    b2p = jnp.zeros((1, D), jnp.float32).at[0, :C].set(b2)

    agg, g, rc = _lin(x, W0.T, b0.reshape(1, D), degp, att[0, 0].reshape(1, 1))
    for i in range(2):
        for k in range(1, 5):
            p = _prop_kernel(g, ei, dummy).reshape(2, N, D)
            ak = att[i, k].reshape(1, 1)
            if k < 4:
                agg, g = _comb(agg, p, rc, ak)
            elif i == 0:
                agg, g = _lin2(agg, p, rc, ak, W1.T, b1.reshape(1, D),
                               att[1, 0].reshape(1, 1))
            else:
                o = _final(agg, p, rc, ak, w2t, b2p)
    return o[:, :C]
